# baseline (speedup 1.0000x reference)
import functools

import jax
import jax.numpy as jnp
from jax import lax
from jax.experimental import pallas as pl
from jax.experimental.pallas import tpu as pltpu
from jax.experimental.pallas import tpu_sc as plsc

B, T, D = 32, 256, 1024
BT = B * T
NSLOT = 4


def _sc_gather_slot0(table_biased, idx_flat):
    info = plsc.get_sparse_core_info()
    nw = info.num_cores * info.num_subcores
    per_w = BT // nw
    chunk = 32
    n_chunks = per_w // chunk

    mesh = plsc.VectorSubcoreMesh(core_axis_name="c", subcore_axis_name="s")

    @functools.partial(
        pl.kernel,
        mesh=mesh,
        out_type=jax.ShapeDtypeStruct((BT, NSLOT * D), jnp.float32),
        scratch_types=[
            pltpu.VMEM((per_w,), jnp.int32),
            pltpu.VMEM((chunk, D), jnp.float32),
            pltpu.VMEM((chunk, D), jnp.float32),
            pltpu.VMEM((chunk, D), jnp.float32),
            pltpu.SemaphoreType.DMA,
            pltpu.SemaphoreType.DMA,
            pltpu.SemaphoreType.DMA,
            pltpu.SemaphoreType.DMA,
            pltpu.SemaphoreType.DMA,
            pltpu.SemaphoreType.DMA,
        ],
    )
    def k(table_hbm, idx_hbm, out_hbm, idx_v, rows_a, rows_b, rows_c,
          gs_a, gs_b, gs_c, ws_a, ws_b, ws_c):
        wid = lax.axis_index("s") * info.num_cores + lax.axis_index("c")
        base = wid * per_w
        pltpu.sync_copy(idx_hbm.at[pl.ds(base, per_w)], idx_v)

        bufs = (rows_a, rows_b, rows_c)
        gsems = (gs_a, gs_b, gs_c)
        wsems = (ws_a, ws_b, ws_c)
        nbuf = 3

        def gather(c):
            return pltpu.async_copy(
                table_hbm.at[idx_v.at[pl.ds(c * chunk, chunk)]],
                bufs[c % nbuf], gsems[c % nbuf])

        def write(c):
            return pltpu.async_copy(
                bufs[c % nbuf],
                out_hbm.at[pl.ds(base + c * chunk, chunk), pl.ds(0, D)],
                wsems[c % nbuf])

        hg = {c: gather(c) for c in range(min(nbuf, n_chunks))}
        hw = {}
        for c in range(n_chunks):
            hg[c].wait()
            hw[c] = write(c)
            if c + nbuf < n_chunks:
                hw[c].wait()
                hg[c + nbuf] = gather(c + nbuf)
        for c in range(max(0, n_chunks - nbuf), n_chunks):
            hw[c].wait()

    return k(table_biased, idx_flat)


BR = 1024


def _tc_dense(tokens0, scroll_r, hotbar_r, buttons, keys, yaw_pitch, gui,
              scroll_table, buttons_W, keys_W, w_yp, w_gui, hotbar_table,
              bias3):
    nb = BT // BR

    def body(alias_ref, scroll_ref, hotbar_ref, btn_ref, keys_ref, yp_ref,
             gui_ref, st_ref, bw_ref, kw_ref, wyp_ref, wgui_ref, ht_ref,
             bias_ref, out_ref):
        f32 = jnp.float32
        sc = scroll_ref[0, 0, :][:, None]
        oh_s = (sc == lax.broadcasted_iota(jnp.int32, (BR, 3), 1)).astype(f32)
        out_ref[:, :D] = (
            jnp.dot(oh_s, st_ref[...], preferred_element_type=f32)
            + jnp.dot(btn_ref[...], bw_ref[...], preferred_element_type=f32)
            + bias_ref[0, 0]
        )
        out_ref[:, D:2 * D] = (
            jnp.dot(keys_ref[...], kw_ref[...], preferred_element_type=f32)
            + bias_ref[1, 0]
        )
        hb = hotbar_ref[0, 0, :][:, None]
        oh_h = (hb == lax.broadcasted_iota(jnp.int32, (BR, 9), 1)).astype(f32)
        out_ref[:, 2 * D:] = (
            jnp.dot(yp_ref[...], wyp_ref[...], preferred_element_type=f32)
            + jnp.dot(gui_ref[...], wgui_ref[...], preferred_element_type=f32)
            + jnp.dot(oh_h, ht_ref[...], preferred_element_type=f32)
            + bias_ref[2, 0]
        )

    full = lambda shape: pl.BlockSpec(shape, lambda b: (0,) * len(shape))
    per_b = lambda shape: pl.BlockSpec(shape, lambda b: (b,) + (0,) * (len(shape) - 1))

    return pl.pallas_call(
        body,
        grid=(nb,),
        in_specs=[
            pl.BlockSpec(memory_space=pl.ANY),
            per_b((1, 1, BR)),
            per_b((1, 1, BR)),
            per_b((BR, 3)),
            per_b((BR, 23)),
            per_b((BR, 2)),
            per_b((BR, 2)),
            full((3, D)),
            full((3, D)),
            full((23, D)),
            full((2, D)),
            full((2, D)),
            full((9, D)),
            full((3, 1, D)),
        ],
        out_specs=pl.BlockSpec((pl.Element(BR), pl.Element(3 * D)),
                               lambda b: (b * BR, D)),
        out_shape=jax.ShapeDtypeStruct((BT, NSLOT * D), jnp.float32),
        input_output_aliases={0: 0},
    )(tokens0, scroll_r, hotbar_r, buttons, keys, yaw_pitch, gui,
      scroll_table, buttons_W, keys_W, w_yp, w_gui, hotbar_table, bias3)


def kernel(mouse_cat, scroll, buttons, keys, yaw_pitch, gui, hotbar,
           mouse_table, scroll_table, hotbar_table, slot_table,
           buttons_W, buttons_b, keys_W, keys_b, yawgui_W, yawgui_b):
    table_biased = mouse_table + slot_table[0][None, :]
    bias3 = jnp.stack([
        slot_table[1] + buttons_b,
        slot_table[2] + keys_b,
        slot_table[3] + yawgui_b,
    ])[:, None, :]
    w_yp = yawgui_W[:2]
    w_gui = yawgui_W[2:]

    idx_flat = mouse_cat.reshape(BT).astype(jnp.int32)
    scroll_r = scroll.reshape(BT // BR, 1, BR).astype(jnp.int32)
    hotbar_r = hotbar.reshape(BT // BR, 1, BR).astype(jnp.int32)

    tokens0 = _sc_gather_slot0(table_biased, idx_flat)
    tokens = _tc_dense(tokens0, scroll_r, hotbar_r,
                       buttons.reshape(BT, 3), keys.reshape(BT, 23),
                       yaw_pitch.reshape(BT, 2), gui.reshape(BT, 2),
                       scroll_table, buttons_W, keys_W,
                       w_yp, w_gui, hotbar_table, bias3)
    return tokens.reshape(B, T, NSLOT, D)
